# Initial kernel scaffold; baseline (speedup 1.0000x reference)
import jax
import jax.numpy as jnp
from jax.experimental import pallas as pl


def kernel(x, neighbor_map, W):
    raise NotImplementedError("write your pallas kernel here")


# speedup vs baseline: 15.7765x; 15.7765x over previous
import functools

import jax
import jax.numpy as jnp
from jax import lax
from jax.experimental import pallas as pl
from jax.experimental.pallas import tpu as pltpu
from jax.experimental.pallas import tpu_sc as plsc

N_PAD = 102400
B = 1024
C = 6400
K_VOL = 27
PLANES = 32


def _tc_prep_kernel(x_ref, nb_ref, w_ref, yT_ref, nbT_ref):
    yT_ref[...] = lax.dot_general(
        w_ref[...], x_ref[...], (((1,), (1,)), ((), ())),
        preferred_element_type=jnp.float32,
        precision=lax.Precision.HIGHEST,
    )
    nbT_ref[...] = jnp.transpose(nb_ref[...], (1, 0))


def _tc_finish_kernel(x_ref, g_ref, out_ref):
    ones = jnp.ones((K_VOL, 1), jnp.float32)
    s = lax.dot_general(
        g_ref[...], ones, (((0,), (0,)), ((), ())),
        preferred_element_type=jnp.float32,
        precision=lax.Precision.HIGHEST,
    )
    out_ref[...] = x_ref[...] * jax.nn.sigmoid(s)


@functools.lru_cache(maxsize=1)
def _make_sc_gather():
    mesh = plsc.VectorSubcoreMesh(core_axis_name="c", subcore_axis_name="s")

    @functools.partial(
        pl.kernel,
        out_type=jax.ShapeDtypeStruct((K_VOL, N_PAD), jnp.float32),
        mesh=mesh,
        compiler_params=pltpu.CompilerParams(needs_layout_passes=False),
        scratch_types=[
            pltpu.VMEM((N_PAD,), jnp.float32),
            pltpu.VMEM((C,), jnp.int32),
            pltpu.VMEM((C,), jnp.float32),
        ],
    )
    def _sc_gather(yT_hbm, nbT_hbm, g_hbm, table_v, idx_v, out_v):
        wid = lax.axis_index("s") * 2 + lax.axis_index("c")

        @pl.when(wid < K_VOL)
        def _():
            pltpu.sync_copy(yT_hbm.at[wid], table_v)

            def chunk_body(ci, carry):
                base = ci * C
                pltpu.sync_copy(nbT_hbm.at[wid, pl.ds(base, C)], idx_v)

                def body(i, carry2):
                    idx = idx_v[pl.ds(i * 16, 16)]
                    out_v[pl.ds(i * 16, 16)] = plsc.load_gather(table_v, [idx])
                    return carry2

                lax.fori_loop(0, C // 16, body, 0)
                pltpu.sync_copy(out_v, g_hbm.at[wid, pl.ds(base, C)])
                return carry

            lax.fori_loop(0, N_PAD // C, chunk_body, 0)

    return _sc_gather


def kernel(x, neighbor_map, W):
    n = x.shape[0]
    xp = jnp.pad(x, ((0, N_PAD - n), (0, 0)))
    nbp = jnp.pad(neighbor_map, ((0, N_PAD - n), (0, 0)))
    wsq = W[:, :, 0]

    yT, nbT = pl.pallas_call(
        _tc_prep_kernel,
        grid=(N_PAD // B,),
        in_specs=[
            pl.BlockSpec((B, PLANES), lambda j: (j, 0)),
            pl.BlockSpec((B, K_VOL), lambda j: (j, 0)),
            pl.BlockSpec((K_VOL, PLANES), lambda j: (0, 0)),
        ],
        out_specs=[
            pl.BlockSpec((K_VOL, B), lambda j: (0, j)),
            pl.BlockSpec((K_VOL, B), lambda j: (0, j)),
        ],
        out_shape=[
            jax.ShapeDtypeStruct((K_VOL, N_PAD), jnp.float32),
            jax.ShapeDtypeStruct((K_VOL, N_PAD), jnp.int32),
        ],
    )(xp, nbp, wsq)

    g = _make_sc_gather()(yT, nbT)

    out = pl.pallas_call(
        _tc_finish_kernel,
        grid=(N_PAD // B,),
        in_specs=[
            pl.BlockSpec((B, PLANES), lambda j: (j, 0)),
            pl.BlockSpec((K_VOL, B), lambda j: (0, j)),
        ],
        out_specs=pl.BlockSpec((B, PLANES), lambda j: (j, 0)),
        out_shape=jax.ShapeDtypeStruct((N_PAD, PLANES), jnp.float32),
    )(xp, g)

    return out[:n]
